# Initial kernel scaffold; baseline (speedup 1.0000x reference)
import jax
import jax.numpy as jnp
from jax.experimental import pallas as pl


def kernel(x, Win, b_in):
    raise NotImplementedError("write your pallas kernel here")


# speedup vs baseline: 2.1867x; 2.1867x over previous
import functools
import jax
import jax.numpy as jnp
from jax.experimental import pallas as pl
from jax.experimental.pallas import tpu as pltpu

_G = 2
_R = 2
_C = 4
_GD = 512
_HALF_L = 2.002
_BASIS = (1.0, 5.0, 25.0, 125.0)
_TT = 512


def _fsq_round(b):
    one = jnp.float32(1.0)
    zero = jnp.float32(0.0)
    q = jnp.where(b > 0.5, one, zero)
    q = q + jnp.where(b >= 1.5, one, zero)
    q = q - jnp.where(b < -0.5, one, zero)
    q = q - jnp.where(b <= -1.5, one, zero)
    return q


def _gfsq_body(x_ref, w_ref, b_ref, o_ref):
    xb = x_ref[0]
    for g in range(_G):
        xg = xb[g * _GD:(g + 1) * _GD, :]
        w = w_ref[g]
        z = jax.lax.dot_general(
            w, xg, (((1,), (0,)), ((), ())),
            preferred_element_type=jnp.float32)
        z = z + b_ref[g][:, None]
        q0 = _fsq_round(jnp.tanh(z) * _HALF_L)
        q1 = _fsq_round(jnp.tanh(4.0 * z - 2.0 * q0) * _HALF_L)
        idx0 = sum((q0[c] + 2.0) * _BASIS[c] for c in range(_C))
        idx1 = sum((q1[c] + 2.0) * _BASIS[c] for c in range(_C))
        o_ref[0, g * _R, :] = idx0.astype(jnp.int32)
        o_ref[0, g * _R + 1, :] = idx1.astype(jnp.int32)


@jax.jit
def kernel(x, Win, b_in):
    B, DIM, T = x.shape
    grid = (B, T // _TT)
    return pl.pallas_call(
        _gfsq_body,
        grid=grid,
        in_specs=[
            pl.BlockSpec((1, DIM, _TT), lambda b, t: (b, 0, t)),
            pl.BlockSpec((_G, _C, _GD), lambda b, t: (0, 0, 0)),
            pl.BlockSpec((_G, _C), lambda b, t: (0, 0)),
        ],
        out_specs=pl.BlockSpec((1, _G * _R, _TT), lambda b, t: (b, 0, t)),
        out_shape=jax.ShapeDtypeStruct((B, _G * _R, T), jnp.int32),
    )(x, Win, b_in)
